# baseline (speedup 1.0000x reference)
import jax
import jax.numpy as jnp
from jax.experimental import pallas as pl
from jax.experimental.pallas import tpu as pltpu

Q = 16
C = 100
DFLAT = 196 * 768
BK = 6272
NK = DFLAT // BK


def _body(q_ref, k_ref, dist_ref, idx_ref, acc_ref):
    kk = pl.program_id(0)

    @pl.when(kk == 0)
    def _init():
        acc_ref[...] = jnp.zeros_like(acc_ref)

    qblk = q_ref[...]
    kblk = k_ref[...]
    qs = jnp.sum(qblk * qblk, axis=1, keepdims=True)
    ks = jnp.sum(kblk * kblk, axis=1)
    dot = jax.lax.dot_general(
        qblk, kblk, (((1,), (1,)), ((), ())),
        preferred_element_type=jnp.float32,
        precision=jax.lax.Precision.HIGHEST,
    )
    acc_ref[...] += qs + ks[None, :] - 2.0 * dot

    @pl.when(kk == NK - 1)
    def _fin():
        d2 = jnp.maximum(acc_ref[...], 0.0)
        idx_ref[...] = jnp.argmin(d2, axis=1, keepdims=True).astype(jnp.int32)
        dist_ref[...] = jnp.sqrt(jnp.min(d2, axis=1, keepdims=True))


def kernel(query_features, keys):
    qf = query_features.reshape(Q, DFLAT)
    kf = keys.reshape(C, DFLAT)
    dist, idx = pl.pallas_call(
        _body,
        grid=(NK,),
        in_specs=[
            pl.BlockSpec((Q, BK), lambda k: (0, k)),
            pl.BlockSpec((C, BK), lambda k: (0, k)),
        ],
        out_specs=[
            pl.BlockSpec((Q, 1), lambda k: (0, 0)),
            pl.BlockSpec((Q, 1), lambda k: (0, 0)),
        ],
        out_shape=[
            jax.ShapeDtypeStruct((Q, 1), jnp.float32),
            jax.ShapeDtypeStruct((Q, 1), jnp.int32),
        ],
        scratch_shapes=[pltpu.VMEM((Q, C), jnp.float32)],
    )(qf, kf)
    return dist.reshape(Q), idx.reshape(Q)


# speedup vs baseline: 2.0441x; 2.0441x over previous
import jax
import jax.numpy as jnp
from jax.experimental import pallas as pl
from jax.experimental.pallas import tpu as pltpu

Q = 16
C = 100
P = 196
D = 768
PB = 8
NFULL = P // PB
TAIL = P - NFULL * PB


def _dot_qk(qp, kp):
    return jax.lax.dot_general(
        qp, kp, (((1,), (1,)), ((), ())),
        preferred_element_type=jnp.float32,
        precision=jax.lax.Precision.HIGHEST,
    )


def _body(q_ref, k_ref, qt_ref, kt_ref, dist_ref, idx_ref,
          acc_ref, s2_ref, q2_ref):
    pp = pl.program_id(0)

    @pl.when(pp == 0)
    def _init():
        acc_ref[...] = jnp.zeros_like(acc_ref)
        s2_ref[...] = jnp.zeros_like(s2_ref)
        q2_ref[...] = jnp.zeros_like(q2_ref)

    dtot = jnp.zeros((Q, C), jnp.float32)
    s2 = jnp.zeros((C, D), jnp.float32)
    q2 = jnp.zeros((Q, D), jnp.float32)
    for p in range(PB):
        qp = q_ref[:, p, :]
        kp = k_ref[:, p, :]
        dtot += _dot_qk(qp, kp)
        s2 += kp * kp
        q2 += qp * qp
    acc_ref[...] += dtot
    s2_ref[...] += s2
    q2_ref[...] += q2

    @pl.when(pp == NFULL - 1)
    def _fin():
        dtail = jnp.zeros((Q, C), jnp.float32)
        s2t = jnp.zeros((C, D), jnp.float32)
        q2t = jnp.zeros((Q, D), jnp.float32)
        for p in range(TAIL):
            qp = qt_ref[:, p, :]
            kp = kt_ref[:, p, :]
            dtail += _dot_qk(qp, kp)
            s2t += kp * kp
            q2t += qp * qp
        ks = jnp.sum(s2_ref[...] + s2t, axis=1)
        qs = jnp.sum(q2_ref[...] + q2t, axis=1, keepdims=True)
        dot = acc_ref[...] + dtail
        d2 = jnp.maximum(qs + ks[None, :] - 2.0 * dot, 0.0)
        idx_ref[...] = jnp.argmin(d2, axis=1, keepdims=True).astype(jnp.int32)
        dist_ref[...] = jnp.sqrt(jnp.min(d2, axis=1, keepdims=True))


def kernel(query_features, keys):
    dist, idx = pl.pallas_call(
        _body,
        grid=(NFULL,),
        in_specs=[
            pl.BlockSpec((Q, PB, D), lambda p: (0, p, 0)),
            pl.BlockSpec((C, PB, D), lambda p: (0, p, 0)),
            pl.BlockSpec((Q, PB, D), lambda p: (0, NFULL, 0)),
            pl.BlockSpec((C, PB, D), lambda p: (0, NFULL, 0)),
        ],
        out_specs=[
            pl.BlockSpec((Q, 1), lambda p: (0, 0)),
            pl.BlockSpec((Q, 1), lambda p: (0, 0)),
        ],
        out_shape=[
            jax.ShapeDtypeStruct((Q, 1), jnp.float32),
            jax.ShapeDtypeStruct((Q, 1), jnp.int32),
        ],
        scratch_shapes=[
            pltpu.VMEM((Q, C), jnp.float32),
            pltpu.VMEM((C, D), jnp.float32),
            pltpu.VMEM((Q, D), jnp.float32),
        ],
    )(query_features, keys, query_features, keys)
    return dist.reshape(Q), idx.reshape(Q)


# speedup vs baseline: 2.7117x; 1.3266x over previous
import jax
import jax.numpy as jnp
from jax.experimental import pallas as pl
from jax.experimental.pallas import tpu as pltpu

Q = 16
C = 100
P = 196
D = 768
PB = 8
NFULL = P // PB
TAIL = P - NFULL * PB
QR = Q * PB
CR = C * PB


def _body(q_ref, k_ref, qt_ref, kt_ref, dist_ref, idx_ref,
          g8_ref, s2_ref, q2_ref):
    pp = pl.program_id(0)

    @pl.when(pp == 0)
    def _init():
        g8_ref[...] = jnp.zeros_like(g8_ref)
        s2_ref[...] = jnp.zeros_like(s2_ref)
        q2_ref[...] = jnp.zeros_like(q2_ref)

    qblk = q_ref[...]
    kblk = k_ref[...]
    qr = qblk.reshape(QR, D)
    kr = kblk.reshape(CR, D)
    qhi = qr.astype(jnp.bfloat16)
    qlo = (qr - qhi.astype(jnp.float32)).astype(jnp.bfloat16)
    khi = kr.astype(jnp.bfloat16)
    klo = (kr - khi.astype(jnp.float32)).astype(jnp.bfloat16)

    def _dot(a, b):
        return jax.lax.dot_general(
            a, b, (((1,), (1,)), ((), ())),
            preferred_element_type=jnp.float32,
        )

    g8_ref[...] += _dot(qhi, khi) + _dot(qhi, klo) + _dot(qlo, khi)
    s2_ref[...] += kblk * kblk
    q2_ref[...] += qblk * qblk

    @pl.when(pp == NFULL - 1)
    def _fin():
        dtail = jnp.zeros((Q, C), jnp.float32)
        s2t = jnp.zeros((C, D), jnp.float32)
        q2t = jnp.zeros((Q, D), jnp.float32)
        for p in range(TAIL):
            qp = qt_ref[:, p, :]
            kp = kt_ref[:, p, :]
            dtail += jax.lax.dot_general(
                qp, kp, (((1,), (1,)), ((), ())),
                preferred_element_type=jnp.float32,
                precision=jax.lax.Precision.HIGHEST,
            )
            s2t += kp * kp
            q2t += qp * qp

        g8 = g8_ref[...]
        row = jax.lax.broadcasted_iota(jnp.int32, (QR, CR), 0)
        col = jax.lax.broadcasted_iota(jnp.int32, (QR, CR), 1)
        g8m = jnp.where((row % PB) == (col % PB), g8, 0.0)
        srow = jax.lax.broadcasted_iota(jnp.int32, (Q, QR), 0)
        scol = jax.lax.broadcasted_iota(jnp.int32, (Q, QR), 1)
        s_fold = jnp.where(srow == scol // PB, 1.0, 0.0)
        frow = jax.lax.broadcasted_iota(jnp.int32, (CR, C), 0)
        fcol = jax.lax.broadcasted_iota(jnp.int32, (CR, C), 1)
        f_fold = jnp.where(frow // PB == fcol, 1.0, 0.0)
        gq = jax.lax.dot_general(
            s_fold, g8m, (((1,), (0,)), ((), ())),
            preferred_element_type=jnp.float32,
            precision=jax.lax.Precision.HIGHEST,
        )
        dot = jax.lax.dot_general(
            gq, f_fold, (((1,), (0,)), ((), ())),
            preferred_element_type=jnp.float32,
            precision=jax.lax.Precision.HIGHEST,
        ) + dtail

        ks = jnp.sum(s2_ref[...], axis=(1, 2)) + jnp.sum(s2t, axis=1)
        qs = (jnp.sum(q2_ref[...], axis=(1, 2))
              + jnp.sum(q2t, axis=1))[:, None]
        d2 = jnp.maximum(qs + ks[None, :] - 2.0 * dot, 0.0)
        idx_ref[...] = jnp.argmin(d2, axis=1, keepdims=True).astype(jnp.int32)
        dist_ref[...] = jnp.sqrt(jnp.min(d2, axis=1, keepdims=True))


def kernel(query_features, keys):
    dist, idx = pl.pallas_call(
        _body,
        grid=(NFULL,),
        in_specs=[
            pl.BlockSpec((Q, PB, D), lambda p: (0, p, 0)),
            pl.BlockSpec((C, PB, D), lambda p: (0, p, 0)),
            pl.BlockSpec((Q, PB, D), lambda p: (0, NFULL, 0)),
            pl.BlockSpec((C, PB, D), lambda p: (0, NFULL, 0)),
        ],
        out_specs=[
            pl.BlockSpec((Q, 1), lambda p: (0, 0)),
            pl.BlockSpec((Q, 1), lambda p: (0, 0)),
        ],
        out_shape=[
            jax.ShapeDtypeStruct((Q, 1), jnp.float32),
            jax.ShapeDtypeStruct((Q, 1), jnp.int32),
        ],
        scratch_shapes=[
            pltpu.VMEM((QR, CR), jnp.float32),
            pltpu.VMEM((C, PB, D), jnp.float32),
            pltpu.VMEM((Q, PB, D), jnp.float32),
        ],
    )(query_features, keys, query_features, keys)
    return dist.reshape(Q), idx.reshape(Q)


# speedup vs baseline: 2.9293x; 1.0803x over previous
import jax
import jax.numpy as jnp
from jax.experimental import pallas as pl
from jax.experimental.pallas import tpu as pltpu

Q = 16
C = 100
P = 196
D = 768
PB = 16
NFULL = P // PB
TAIL = P - NFULL * PB
QR = Q * PB
CR = C * PB


def _body(q_ref, k_ref, qt_ref, kt_ref, dist_ref, idx_ref,
          g8_ref, s2_ref, q2_ref):
    pp = pl.program_id(0)

    @pl.when(pp == 0)
    def _init():
        g8_ref[...] = jnp.zeros_like(g8_ref)
        s2_ref[...] = jnp.zeros_like(s2_ref)
        q2_ref[...] = jnp.zeros_like(q2_ref)

    qblk = q_ref[...]
    kblk = k_ref[...]
    qr = qblk.reshape(QR, D)
    kr = kblk.reshape(CR, D)
    qhi = qr.astype(jnp.bfloat16)
    qlo = (qr - qhi.astype(jnp.float32)).astype(jnp.bfloat16)
    khi = kr.astype(jnp.bfloat16)
    klo = (kr - khi.astype(jnp.float32)).astype(jnp.bfloat16)

    def _dot(a, b):
        return jax.lax.dot_general(
            a, b, (((1,), (1,)), ((), ())),
            preferred_element_type=jnp.float32,
        )

    g8_ref[...] += _dot(qhi, khi) + _dot(qhi, klo) + _dot(qlo, khi)
    s2_ref[...] += kblk * kblk
    q2_ref[...] += qblk * qblk

    @pl.when(pp == NFULL - 1)
    def _fin():
        dtail = jnp.zeros((Q, C), jnp.float32)
        s2t = jnp.zeros((C, D), jnp.float32)
        q2t = jnp.zeros((Q, D), jnp.float32)
        for p in range(TAIL):
            qp = qt_ref[:, p, :]
            kp = kt_ref[:, p, :]
            dtail += jax.lax.dot_general(
                qp, kp, (((1,), (1,)), ((), ())),
                preferred_element_type=jnp.float32,
                precision=jax.lax.Precision.HIGHEST,
            )
            s2t += kp * kp
            q2t += qp * qp

        g8 = g8_ref[...]
        row = jax.lax.broadcasted_iota(jnp.int32, (QR, CR), 0)
        col = jax.lax.broadcasted_iota(jnp.int32, (QR, CR), 1)
        g8m = jnp.where((row % PB) == (col % PB), g8, 0.0)
        srow = jax.lax.broadcasted_iota(jnp.int32, (Q, QR), 0)
        scol = jax.lax.broadcasted_iota(jnp.int32, (Q, QR), 1)
        s_fold = jnp.where(srow == scol // PB, 1.0, 0.0)
        frow = jax.lax.broadcasted_iota(jnp.int32, (CR, C), 0)
        fcol = jax.lax.broadcasted_iota(jnp.int32, (CR, C), 1)
        f_fold = jnp.where(frow // PB == fcol, 1.0, 0.0)
        gq = jax.lax.dot_general(
            s_fold, g8m, (((1,), (0,)), ((), ())),
            preferred_element_type=jnp.float32,
            precision=jax.lax.Precision.HIGHEST,
        )
        dot = jax.lax.dot_general(
            gq, f_fold, (((1,), (0,)), ((), ())),
            preferred_element_type=jnp.float32,
            precision=jax.lax.Precision.HIGHEST,
        ) + dtail

        ks = jnp.sum(s2_ref[...], axis=(1, 2)) + jnp.sum(s2t, axis=1)
        qs = (jnp.sum(q2_ref[...], axis=(1, 2))
              + jnp.sum(q2t, axis=1))[:, None]
        d2 = jnp.maximum(qs + ks[None, :] - 2.0 * dot, 0.0)
        idx_ref[...] = jnp.argmin(d2, axis=1, keepdims=True).astype(jnp.int32)
        dist_ref[...] = jnp.sqrt(jnp.min(d2, axis=1, keepdims=True))


def kernel(query_features, keys):
    dist, idx = pl.pallas_call(
        _body,
        grid=(NFULL,),
        in_specs=[
            pl.BlockSpec((Q, PB, D), lambda p: (0, p, 0)),
            pl.BlockSpec((C, PB, D), lambda p: (0, p, 0)),
            pl.BlockSpec((Q, PB, D), lambda p: (0, NFULL, 0)),
            pl.BlockSpec((C, PB, D), lambda p: (0, NFULL, 0)),
        ],
        out_specs=[
            pl.BlockSpec((Q, 1), lambda p: (0, 0)),
            pl.BlockSpec((Q, 1), lambda p: (0, 0)),
        ],
        out_shape=[
            jax.ShapeDtypeStruct((Q, 1), jnp.float32),
            jax.ShapeDtypeStruct((Q, 1), jnp.int32),
        ],
        scratch_shapes=[
            pltpu.VMEM((QR, CR), jnp.float32),
            pltpu.VMEM((C, PB, D), jnp.float32),
            pltpu.VMEM((Q, PB, D), jnp.float32),
        ],
    )(query_features, keys, query_features, keys)
    return dist.reshape(Q), idx.reshape(Q)


# speedup vs baseline: 3.0358x; 1.0364x over previous
import jax
import jax.numpy as jnp
from jax.experimental import pallas as pl
from jax.experimental.pallas import tpu as pltpu

Q = 16
C = 100
P = 196
D = 768
PB = 16
NFULL = P // PB
TAIL = P - NFULL * PB
QR = Q * PB
CR = C * PB


def _body(q_ref, k_ref, dist_ref, idx_ref, g8_ref, s2_ref, q2_ref):
    pp = pl.program_id(0)

    @pl.when(pp == 0)
    def _init():
        g8_ref[...] = jnp.zeros_like(g8_ref)
        s2_ref[...] = jnp.zeros_like(s2_ref)
        q2_ref[...] = jnp.zeros_like(q2_ref)

    @pl.when(pp < NFULL)
    def _main():
        qblk = q_ref[...]
        kblk = k_ref[...]
        qr = qblk.reshape(QR, D)
        kr = kblk.reshape(CR, D)
        qhi = qr.astype(jnp.bfloat16)
        qlo = (qr - qhi.astype(jnp.float32)).astype(jnp.bfloat16)
        khi = kr.astype(jnp.bfloat16)
        klo = (kr - khi.astype(jnp.float32)).astype(jnp.bfloat16)

        def _dot(a, b):
            return jax.lax.dot_general(
                a, b, (((1,), (1,)), ((), ())),
                preferred_element_type=jnp.float32,
            )

        g8_ref[...] += _dot(qhi, khi) + _dot(qhi, klo) + _dot(qlo, khi)
        s2_ref[...] += kblk * kblk
        q2_ref[...] += qblk * qblk

    @pl.when(pp == NFULL)
    def _fin():
        dtail = jnp.zeros((Q, C), jnp.float32)
        s2t = jnp.zeros((C, D), jnp.float32)
        q2t = jnp.zeros((Q, D), jnp.float32)
        for p in range(TAIL):
            qp = q_ref[:, p, :]
            kp = k_ref[:, p, :]
            dtail += jax.lax.dot_general(
                qp, kp, (((1,), (1,)), ((), ())),
                preferred_element_type=jnp.float32,
                precision=jax.lax.Precision.HIGHEST,
            )
            s2t += kp * kp
            q2t += qp * qp

        g8 = g8_ref[...]
        row = jax.lax.broadcasted_iota(jnp.int32, (QR, CR), 0)
        col = jax.lax.broadcasted_iota(jnp.int32, (QR, CR), 1)
        g8m = jnp.where((row % PB) == (col % PB), g8, 0.0)
        srow = jax.lax.broadcasted_iota(jnp.int32, (Q, QR), 0)
        scol = jax.lax.broadcasted_iota(jnp.int32, (Q, QR), 1)
        s_fold = jnp.where(srow == scol // PB, 1.0, 0.0)
        frow = jax.lax.broadcasted_iota(jnp.int32, (CR, C), 0)
        fcol = jax.lax.broadcasted_iota(jnp.int32, (CR, C), 1)
        f_fold = jnp.where(frow // PB == fcol, 1.0, 0.0)
        gq = jax.lax.dot_general(
            s_fold, g8m, (((1,), (0,)), ((), ())),
            preferred_element_type=jnp.float32,
            precision=jax.lax.Precision.HIGHEST,
        )
        dot = jax.lax.dot_general(
            gq, f_fold, (((1,), (0,)), ((), ())),
            preferred_element_type=jnp.float32,
            precision=jax.lax.Precision.HIGHEST,
        ) + dtail

        ks = jnp.sum(s2_ref[...], axis=(1, 2)) + jnp.sum(s2t, axis=1)
        qs = (jnp.sum(q2_ref[...], axis=(1, 2))
              + jnp.sum(q2t, axis=1))[:, None]
        d2 = jnp.maximum(qs + ks[None, :] - 2.0 * dot, 0.0)
        idx_ref[...] = jnp.argmin(d2, axis=1, keepdims=True).astype(jnp.int32)
        dist_ref[...] = jnp.sqrt(jnp.min(d2, axis=1, keepdims=True))


def kernel(query_features, keys):
    dist, idx = pl.pallas_call(
        _body,
        grid=(NFULL + 1,),
        in_specs=[
            pl.BlockSpec((Q, PB, D), lambda p: (0, p, 0)),
            pl.BlockSpec((C, PB, D), lambda p: (0, p, 0)),
        ],
        out_specs=[
            pl.BlockSpec((Q, 1), lambda p: (0, 0)),
            pl.BlockSpec((Q, 1), lambda p: (0, 0)),
        ],
        out_shape=[
            jax.ShapeDtypeStruct((Q, 1), jnp.float32),
            jax.ShapeDtypeStruct((Q, 1), jnp.int32),
        ],
        scratch_shapes=[
            pltpu.VMEM((QR, CR), jnp.float32),
            pltpu.VMEM((C, PB, D), jnp.float32),
            pltpu.VMEM((Q, PB, D), jnp.float32),
        ],
    )(query_features, keys)
    return dist.reshape(Q), idx.reshape(Q)


# speedup vs baseline: 4.0619x; 1.3380x over previous
import jax
import jax.numpy as jnp
from jax.experimental import pallas as pl
from jax.experimental.pallas import tpu as pltpu

Q = 16
C = 100
P = 196
D = 768
PB = 16
NFULL = P // PB
TAIL = P - NFULL * PB
QR = Q * PB
CR = C * PB


def _body(q_ref, k_ref, dist_ref, idx_ref, g8_ref, s2_ref, q2_ref):
    pp = pl.program_id(0)

    @pl.when(pp == 0)
    def _init():
        g8_ref[...] = jnp.zeros_like(g8_ref)
        s2_ref[...] = jnp.zeros_like(s2_ref)
        q2_ref[...] = jnp.zeros_like(q2_ref)

    @pl.when(pp < NFULL)
    def _main():
        qblk = q_ref[...]
        kblk = k_ref[...]
        qr = qblk.reshape(QR, D)
        kr = kblk.reshape(CR, D)
        qhi = qr.astype(jnp.bfloat16)
        qlo = (qr - qhi.astype(jnp.float32)).astype(jnp.bfloat16)
        khi = kr.astype(jnp.bfloat16)
        klo = (kr - khi.astype(jnp.float32)).astype(jnp.bfloat16)

        def _dot(a, b):
            return jax.lax.dot_general(
                a, b, (((1,), (1,)), ((), ())),
                preferred_element_type=jnp.float32,
            )

        g8_ref[...] += _dot(qhi, khi) + _dot(qhi, klo) + _dot(qlo, khi)
        s2_ref[...] += kblk * kblk
        q2_ref[...] += qblk * qblk

    @pl.when(pp == NFULL)
    def _fin():
        dtail = jnp.zeros((Q, C), jnp.float32)
        s2t = jnp.zeros((C, D), jnp.float32)
        q2t = jnp.zeros((Q, D), jnp.float32)
        for p in range(TAIL):
            qp = q_ref[p, :, :]
            kp = k_ref[:, p, :]
            dtail += jax.lax.dot_general(
                qp, kp, (((1,), (1,)), ((), ())),
                preferred_element_type=jnp.float32,
                precision=jax.lax.Precision.HIGHEST,
            )
            s2t += kp * kp
            q2t += qp * qp

        g8 = g8_ref[...]
        row = jax.lax.broadcasted_iota(jnp.int32, (QR, CR), 0)
        col = jax.lax.broadcasted_iota(jnp.int32, (QR, CR), 1)
        g8m = jnp.where((row // Q) == (col % PB), g8, 0.0)
        srow = jax.lax.broadcasted_iota(jnp.int32, (Q, QR), 0)
        scol = jax.lax.broadcasted_iota(jnp.int32, (Q, QR), 1)
        s_fold = jnp.where(srow == scol % Q, 1.0, 0.0)
        frow = jax.lax.broadcasted_iota(jnp.int32, (CR, C), 0)
        fcol = jax.lax.broadcasted_iota(jnp.int32, (CR, C), 1)
        f_fold = jnp.where(frow // PB == fcol, 1.0, 0.0)
        gq = jax.lax.dot_general(
            s_fold, g8m, (((1,), (0,)), ((), ())),
            preferred_element_type=jnp.float32,
            precision=jax.lax.Precision.HIGHEST,
        )
        dot = jax.lax.dot_general(
            gq, f_fold, (((1,), (0,)), ((), ())),
            preferred_element_type=jnp.float32,
            precision=jax.lax.Precision.HIGHEST,
        ) + dtail

        ks = jnp.sum(s2_ref[...], axis=(1, 2)) + jnp.sum(s2t, axis=1)
        qs = (jnp.sum(q2_ref[...], axis=(0, 2))
              + jnp.sum(q2t, axis=1))[:, None]
        d2 = jnp.maximum(qs + ks[None, :] - 2.0 * dot, 0.0)
        idx_ref[...] = jnp.argmin(d2, axis=1).astype(jnp.int32)
        dist_ref[...] = jnp.sqrt(jnp.min(d2, axis=1))


def kernel(query_features, keys):
    qt = jnp.swapaxes(query_features, 0, 1)
    dist, idx = pl.pallas_call(
        _body,
        grid=(NFULL + 1,),
        in_specs=[
            pl.BlockSpec((PB, Q, D), lambda p: (p, 0, 0)),
            pl.BlockSpec((C, PB, D), lambda p: (0, p, 0)),
        ],
        out_specs=[
            pl.BlockSpec((Q,), lambda p: (0,)),
            pl.BlockSpec((Q,), lambda p: (0,)),
        ],
        out_shape=[
            jax.ShapeDtypeStruct((Q,), jnp.float32),
            jax.ShapeDtypeStruct((Q,), jnp.int32),
        ],
        scratch_shapes=[
            pltpu.VMEM((QR, CR), jnp.float32),
            pltpu.VMEM((C, PB, D), jnp.float32),
            pltpu.VMEM((PB, Q, D), jnp.float32),
        ],
    )(qt, keys)
    return dist, idx
